# baseline (speedup 1.0000x reference)
import functools

import jax
import jax.numpy as jnp
from jax import lax
from jax.experimental import pallas as pl
from jax.experimental.pallas import tpu as pltpu
from jax.experimental.pallas import tpu_sc as plsc

NN = 10000
EE = 320000
HH = 128
NBLK = 3
NHID = 3

NW = 32
EPW = EE // NW
CH = 80
NCH = EPW // CH
NROW = EE // CH
NN2 = 10240
RPT = NN2 // 16
LB = HH // 16

SEG = 5
CPS = NCH // SEG

BE = 4000
GB = EE // BE


@functools.lru_cache(maxsize=None)
def _sc_kernels():
    mesh = plsc.VectorSubcoreMesh(core_axis_name="c", subcore_axis_name="s")

    @functools.partial(
        pl.kernel,
        out_type=jax.ShapeDtypeStruct((EE, HH), jnp.float32),
        mesh=mesh,
        compiler_params=pltpu.CompilerParams(needs_layout_passes=False),
        scratch_types=[
            pltpu.VMEM((NCH, CH), jnp.int32),
            pltpu.VMEM((NCH, CH), jnp.int32),
            pltpu.VMEM((3, CH, HH), jnp.float32),
            pltpu.VMEM((3, CH, HH), jnp.float32),
            pltpu.SemaphoreType.DMA,
            pltpu.SemaphoreType.DMA,
        ],
    )
    def _gather_sum(p_hbm, q_hbm, dst_hbm, src_hbm, x1_hbm,
                    dst_v, src_v, rp_v, rq_v, gsem, wsem):
        wid = lax.axis_index("s") * 2 + lax.axis_index("c")
        ebase = wid * EPW
        pltpu.sync_copy(dst_hbm.at[wid], dst_v)
        pltpu.sync_copy(src_hbm.at[wid], src_v)

        def start(j, p):
            pltpu.async_copy(p_hbm.at[dst_v.at[j]], rp_v.at[p], gsem)
            pltpu.async_copy(q_hbm.at[src_v.at[j]], rq_v.at[p], gsem)

        def wait_g(j, p):
            pltpu.make_async_copy(p_hbm.at[dst_v.at[j]], rp_v.at[p],
                                  gsem).wait()
            pltpu.make_async_copy(q_hbm.at[src_v.at[j]], rq_v.at[p],
                                  gsem).wait()

        def wait_w(j, p):
            pltpu.make_async_copy(
                rp_v.at[p], x1_hbm.at[pl.ds(ebase + j * CH, CH)], wsem).wait()

        start(0, 0)
        start(1, 1)

        def chunk(j, carry):
            p = j % 3
            wait_g(j, p)

            @pl.when(j + 2 < NCH)
            def _():
                @pl.when(j >= 1)
                def _():
                    wait_w(j - 1, (j + 2) % 3)

                start(j + 2, (j + 2) % 3)

            def row(i, c2):
                for c in range(LB):
                    sl = pl.ds(c * 16, 16)
                    rp_v[p, i, sl] = rp_v[p, i, sl] + rq_v[p, i, sl]
                return c2

            lax.fori_loop(0, CH, row, 0, unroll=4)
            pltpu.async_copy(rp_v.at[p], x1_hbm.at[pl.ds(ebase + j * CH, CH)],
                             wsem)
            return carry

        lax.fori_loop(0, NCH, chunk, 0)
        wait_w(NCH - 3, (NCH - 3) % 3)
        wait_w(NCH - 2, (NCH - 2) % 3)
        wait_w(NCH - 1, (NCH - 1) % 3)

    @functools.partial(
        pl.kernel,
        out_type=jax.ShapeDtypeStruct((NW, NN2), jnp.float32),
        mesh=mesh,
        compiler_params=pltpu.CompilerParams(needs_layout_passes=False),
        scratch_types=[
            pltpu.VMEM((NCH, CH), jnp.int32),
            pltpu.VMEM((NCH, CH), jnp.float32),
            pltpu.VMEM((NN2,), jnp.float32),
        ],
    )
    def _scatter_rep(alpha_hbm, dst_hbm, big_hbm, cands_hbm, dst_v, al_v,
                     cand_v):
        wid = lax.axis_index("s") * 2 + lax.axis_index("c")
        pltpu.sync_copy(big_hbm, cand_v)
        pltpu.sync_copy(dst_hbm.at[wid], dst_v)
        pltpu.sync_copy(alpha_hbm.at[wid], al_v)

        def chunk(j, carry):
            def ek(k, c2):
                sl = pl.ds(k * 16, 16)
                idx = dst_v[j, sl]
                plsc.store_scatter(cand_v, [idx], al_v[j, sl])
                return c2

            lax.fori_loop(0, CH // 16, ek, 0)
            return carry

        lax.fori_loop(0, NCH, chunk, 0)
        pltpu.sync_copy(cand_v, cands_hbm.at[wid])

    RR = 1024
    @functools.partial(
        pl.kernel,
        out_type=jax.ShapeDtypeStruct((NN2,), jnp.float32),
        mesh=mesh,
        compiler_params=pltpu.CompilerParams(needs_layout_passes=False),
        scratch_types=[
            pltpu.VMEM((NW, RR), jnp.float32),
        ],
    )
    def _reduce_rep(cands_hbm, mhat_hbm, r_v):
        wid = lax.axis_index("s") * 2 + lax.axis_index("c")

        @pl.when(wid < NN2 // RR)
        def _():
            cols = pl.ds(wid * RR, RR)
            pltpu.sync_copy(cands_hbm.at[:, cols], r_v)

            def red(w, carry):
                def vm(v, c2):
                    sl = pl.ds(v * 16, 16)
                    r_v[0, sl] = jnp.minimum(r_v[0, sl], r_v[w, sl])
                    return c2

                lax.fori_loop(0, RR // 16, vm, 0)
                return carry

            lax.fori_loop(1, NW, red, 0)
            pltpu.sync_copy(r_v.at[0], mhat_hbm.at[cols])

    @functools.partial(
        pl.kernel,
        out_type=(
            jax.ShapeDtypeStruct((NW, NCH * CH), jnp.float32),
            jax.ShapeDtypeStruct((NW, NN), jnp.float32),
        ),
        mesh=mesh,
        compiler_params=pltpu.CompilerParams(needs_layout_passes=False),
        scratch_types=[
            pltpu.VMEM((NCH, CH), jnp.int32),
            pltpu.VMEM((NCH, CH), jnp.float32),
            pltpu.VMEM((NN2,), jnp.float32),
            pltpu.VMEM((NN,), jnp.float32),
            pltpu.VMEM((NCH * CH,), jnp.float32),
        ],
    )
    def _edge_weights(alpha_hbm, dst_hbm, mhat_hbm, zs_hbm, e_hbm, s_hbm,
                      dst_v, al_v, mh_v, s_v, e_v):
        wid = lax.axis_index("s") * 2 + lax.axis_index("c")
        pltpu.sync_copy(zs_hbm, s_v)
        pltpu.sync_copy(mhat_hbm, mh_v)
        pltpu.sync_copy(dst_hbm.at[wid], dst_v)
        pltpu.sync_copy(alpha_hbm.at[wid], al_v)

        def chunk(j, carry):
            def ek(k, c2):
                sl = pl.ds(k * 16, 16)
                idx = dst_v[j, sl]
                m = plsc.load_gather(mh_v, [idx])
                e = jnp.exp(al_v[j, sl] - m)
                e_v[pl.ds(j * CH + k * 16, 16)] = e
                plsc.addupdate_scatter(s_v, [idx], e)
                return c2

            lax.fori_loop(0, CH // 16, ek, 0, unroll=5)
            return carry

        lax.fori_loop(0, NCH, chunk, 0)
        pltpu.sync_copy(e_v, e_hbm.at[wid])
        pltpu.sync_copy(s_v, s_hbm.at[wid])

    @functools.partial(
        pl.kernel,
        out_type=jax.ShapeDtypeStruct((2, NN2, HH), jnp.float32),
        mesh=mesh,
        compiler_params=pltpu.CompilerParams(needs_layout_passes=False),
        scratch_types=[
            pltpu.VMEM((CPS, CH), jnp.int32),
            pltpu.VMEM((CPS, CH), jnp.int32),
            pltpu.VMEM((NCH * CH,), jnp.float32),
            pltpu.VMEM((2, CH, HH), jnp.float32),
            pltpu.VMEM_SHARED((NN2, HH), jnp.float32),
            pltpu.SemaphoreType.DMA,
            pltpu.SemaphoreType.DMA,
        ],
    )
    def _edge_softmax_scatter(e4_hbm, dst_hbm, src_hbm, feat_hbm,
                              zrows_hbm, outp_hbm,
                              dst_v, src_v, e_v, rows_v,
                              out_sh, gsem, wsem):
        cid = lax.axis_index("c")
        sid = lax.axis_index("s")
        wid = sid * 2 + cid
        myrows = pl.ds(sid * RPT, RPT)

        pltpu.sync_copy(zrows_hbm, rows_v.at[0])

        def zinit(w, carry):
            pltpu.sync_copy(rows_v.at[0],
                            out_sh.at[pl.ds(sid * RPT + w * CH, CH)])
            return carry

        lax.fori_loop(0, RPT // CH, zinit, 0)
        pltpu.sync_copy(e4_hbm.at[wid], e_v)
        plsc.subcore_barrier()

        def seg(g, carry0):
            pltpu.sync_copy(dst_hbm.at[wid, g], dst_v)
            pltpu.sync_copy(src_hbm.at[wid, g], src_v)

            def start(j, p):
                pltpu.async_copy(feat_hbm.at[src_v.at[j]], rows_v.at[p], gsem)

            def wait_g(j, p):
                pltpu.make_async_copy(feat_hbm.at[src_v.at[j]], rows_v.at[p],
                                      gsem).wait()

            def wait_s(j, p):
                pltpu.make_async_copy(rows_v.at[p],
                                      out_sh.at[dst_v.at[j]], wsem).wait()

            start(0, 0)

            def chunk(j, carry):
                p = j % 2
                wait_g(j, p)

                @pl.when(j + 1 < CPS)
                def _():
                    @pl.when(j >= 1)
                    def _():
                        wait_s(j - 1, 1 - p)

                    start(j + 1, 1 - p)

                def row(i, c2):
                    eb = plsc.load_gather(
                        e_v, [jnp.full((16,), (g * CPS + j) * CH + i,
                                       jnp.int32)])
                    for c in range(LB):
                        sl = pl.ds(c * 16, 16)
                        rows_v[p, i, sl] = rows_v[p, i, sl] * eb
                    return c2

                lax.fori_loop(0, CH, row, 0, unroll=4)
                pltpu.async_copy(rows_v.at[p], out_sh.at[dst_v.at[j]],
                                 wsem, add=True)
                return carry

            lax.fori_loop(0, CPS, chunk, 0)
            wait_s(CPS - 2, (CPS - 2) % 2)
            wait_s(CPS - 1, (CPS - 1) % 2)
            return carry0

        lax.fori_loop(0, SEG, seg, 0)
        plsc.subcore_barrier()
        pltpu.sync_copy(out_sh.at[myrows], outp_hbm.at[cid, myrows])

    return (_gather_sum, _scatter_rep, _reduce_rep, _edge_weights,
            _edge_softmax_scatter)


def _upd_proj_body(feat_ref, outp_ref, s_ref, w1_ref, w2_ref,
                   featn_ref, p_ref, q_ref):
    s = jnp.sum(s_ref[...], axis=(0, 1, 2))
    inv = 1.0 / (s + 1e-16)
    fn = feat_ref[...] + (outp_ref[0] + outp_ref[1]) * inv[:, None]
    featn_ref[...] = fn
    p_ref[...] = jnp.dot(fn, w1_ref[...], preferred_element_type=jnp.float32)
    q_ref[...] = jnp.dot(fn, w2_ref[...], preferred_element_type=jnp.float32)


_RB = 1000


def _upd_proj(feat, outp, s_all, w1, w2):
    bs_rows = pl.BlockSpec((_RB, HH), lambda g: (g, 0))
    return pl.pallas_call(
        _upd_proj_body,
        grid=(NN // _RB,),
        in_specs=[
            bs_rows,
            pl.BlockSpec((2, _RB, HH), lambda g: (0, g, 0)),
            pl.BlockSpec((NW, 1, 1, _RB), lambda g: (0, g, 0, 0)),
            pl.BlockSpec((HH, HH), lambda g: (0, 0)),
            pl.BlockSpec((HH, HH), lambda g: (0, 0)),
        ],
        out_specs=[bs_rows, bs_rows, bs_rows],
        out_shape=[
            jax.ShapeDtypeStruct((NN, HH), jnp.float32),
            jax.ShapeDtypeStruct((NN, HH), jnp.float32),
            jax.ShapeDtypeStruct((NN, HH), jnp.float32),
        ],
    )(feat, outp, s_all.reshape(NW, NN // _RB, 1, _RB), w1, w2)


def _leaky(x):
    return jnp.where(x > 0, x, 0.01 * x)


def _mlp_body(x1_ref, geo_ref, w3_ref, bin_ref, wh_ref, bh_ref,
              att_ref, alpha_ref):
    h = (x1_ref[...] + jnp.dot(geo_ref[...],
                               w3_ref[...].astype(jnp.bfloat16),
                               preferred_element_type=jnp.float32)
         + bin_ref[...])
    h = jnp.maximum(h, 0.0)
    for l in range(NHID):
        h = jnp.maximum(
            jnp.dot(h, wh_ref[l], preferred_element_type=jnp.float32)
            + bh_ref[l], 0.0)
    h = jnp.maximum(h, 0.01 * h)
    u = h * att_ref[...]
    t = jnp.maximum(u, 0.01 * u)
    ones = jnp.ones((HH, 1), jnp.float32)
    alpha_ref[0, 0, :] = jnp.dot(t, ones,
                                 preferred_element_type=jnp.float32)[:, 0]


def _mlp(x1, geo, w3, b_in, w_hid, b_hid, att):
    bs_edge = pl.BlockSpec((BE, HH), lambda g: (g, 0))
    bs_w = pl.BlockSpec((HH, HH), lambda g: (0, 0))
    alpha3 = pl.pallas_call(
        _mlp_body,
        grid=(GB,),
        in_specs=[
            bs_edge,
            bs_edge,
            bs_w,
            pl.BlockSpec((1, HH), lambda g: (0, 0)),
            pl.BlockSpec((NHID, HH, HH), lambda g: (0, 0, 0)),
            pl.BlockSpec((NHID, HH), lambda g: (0, 0)),
            pl.BlockSpec((1, HH), lambda g: (0, 0)),
        ],
        out_specs=pl.BlockSpec((1, 1, BE), lambda g: (g, 0, 0)),
        out_shape=jax.ShapeDtypeStruct((GB, 1, BE), jnp.float32),
    )(x1, geo, w3, b_in, w_hid, b_hid, att)
    return alpha3.reshape(EE)


def kernel(node_feature, geo_encoding, edge_index, W_in, b_in, W_hid, b_hid, att):
    (gather_sum, scatter_rep, reduce_rep, edge_weights,
     edge_softmax_scatter) = _sc_kernels()
    src2 = edge_index[0].reshape(NW, NCH, CH)
    dst2 = edge_index[1].reshape(NW, NCH, CH)
    zrows = jnp.zeros((CH, HH), jnp.float32)
    zs = jnp.zeros((NN,), jnp.float32)
    big = jnp.full((NN2,), 3e38, jnp.float32)
    outp = jnp.zeros((2, NN2, HH), jnp.float32)
    s_all = jnp.zeros((NW, NN), jnp.float32)

    geo_bf = geo_encoding.astype(jnp.bfloat16)
    feat = node_feature
    for blk in range(NBLK):
        feat, p, q = _upd_proj(feat, outp, s_all,
                               W_in[blk, :HH], W_in[blk, HH:2 * HH])
        x1 = gather_sum(p, q, dst2, src2)
        alpha = _mlp(x1, geo_bf, W_in[blk, 2 * HH:],
                     b_in[blk].reshape(1, HH), W_hid[blk], b_hid[blk],
                     att[blk].reshape(1, HH))
        cands = scatter_rep(alpha.reshape(NW, NCH, CH), dst2, big)
        mhat = reduce_rep(cands)
        e4, s_all = edge_weights(alpha.reshape(NW, NCH, CH), dst2, mhat, zs)
        outp = edge_softmax_scatter(
            e4, dst2.reshape(NW, SEG, CPS, CH),
            src2.reshape(NW, SEG, CPS, CH), feat, zrows)
    feat, _, _ = _upd_proj(feat, outp, s_all, W_in[0, :HH], W_in[0, HH:2 * HH])
    return feat


# speedup vs baseline: 1.0117x; 1.0117x over previous
import functools

import jax
import jax.numpy as jnp
from jax import lax
from jax.experimental import pallas as pl
from jax.experimental.pallas import tpu as pltpu
from jax.experimental.pallas import tpu_sc as plsc

NN = 10000
EE = 320000
HH = 128
NBLK = 3
NHID = 3

NW = 32
EPW = EE // NW
CH = 80
NCH = EPW // CH
NROW = EE // CH
NN2 = 10240
RPT = NN2 // 16
LB = HH // 16

SEG = 5
CPS = NCH // SEG

BE = 4000
GB = EE // BE


@functools.lru_cache(maxsize=None)
def _sc_kernels():
    mesh = plsc.VectorSubcoreMesh(core_axis_name="c", subcore_axis_name="s")

    @functools.partial(
        pl.kernel,
        out_type=jax.ShapeDtypeStruct((EE, HH), jnp.float32),
        mesh=mesh,
        compiler_params=pltpu.CompilerParams(needs_layout_passes=False),
        scratch_types=[
            pltpu.VMEM((NCH, CH), jnp.int32),
            pltpu.VMEM((NCH, CH), jnp.int32),
            pltpu.VMEM((3, CH, HH), jnp.float32),
            pltpu.VMEM((3, CH, HH), jnp.float32),
            pltpu.SemaphoreType.DMA,
            pltpu.SemaphoreType.DMA,
        ],
    )
    def _gather_sum(p_hbm, q_hbm, dst_hbm, src_hbm, x1_hbm,
                    dst_v, src_v, rp_v, rq_v, gsem, wsem):
        wid = lax.axis_index("s") * 2 + lax.axis_index("c")
        ebase = wid * EPW
        pltpu.sync_copy(dst_hbm.at[wid], dst_v)
        pltpu.sync_copy(src_hbm.at[wid], src_v)

        def start(j, p):
            pltpu.async_copy(p_hbm.at[dst_v.at[j]], rp_v.at[p], gsem)
            pltpu.async_copy(q_hbm.at[src_v.at[j]], rq_v.at[p], gsem)

        def wait_g(j, p):
            pltpu.make_async_copy(p_hbm.at[dst_v.at[j]], rp_v.at[p],
                                  gsem).wait()
            pltpu.make_async_copy(q_hbm.at[src_v.at[j]], rq_v.at[p],
                                  gsem).wait()

        def wait_w(j, p):
            pltpu.make_async_copy(
                rp_v.at[p], x1_hbm.at[pl.ds(ebase + j * CH, CH)], wsem).wait()

        start(0, 0)
        start(1, 1)

        def chunk(j, carry):
            p = j % 3
            wait_g(j, p)

            @pl.when(j + 2 < NCH)
            def _():
                @pl.when(j >= 1)
                def _():
                    wait_w(j - 1, (j + 2) % 3)

                start(j + 2, (j + 2) % 3)

            def row(i, c2):
                for c in range(LB):
                    sl = pl.ds(c * 16, 16)
                    rp_v[p, i, sl] = rp_v[p, i, sl] + rq_v[p, i, sl]
                return c2

            lax.fori_loop(0, CH, row, 0, unroll=4)
            pltpu.async_copy(rp_v.at[p], x1_hbm.at[pl.ds(ebase + j * CH, CH)],
                             wsem)
            return carry

        lax.fori_loop(0, NCH, chunk, 0)
        wait_w(NCH - 3, (NCH - 3) % 3)
        wait_w(NCH - 2, (NCH - 2) % 3)
        wait_w(NCH - 1, (NCH - 1) % 3)

    @functools.partial(
        pl.kernel,
        out_type=jax.ShapeDtypeStruct((NW, NN2), jnp.float32),
        mesh=mesh,
        compiler_params=pltpu.CompilerParams(needs_layout_passes=False),
        scratch_types=[
            pltpu.VMEM((NCH, CH), jnp.int32),
            pltpu.VMEM((NCH, CH), jnp.float32),
            pltpu.VMEM((NN2,), jnp.float32),
        ],
    )
    def _scatter_rep(alpha_hbm, dst_hbm, big_hbm, cands_hbm, dst_v, al_v,
                     cand_v):
        wid = lax.axis_index("s") * 2 + lax.axis_index("c")
        pltpu.sync_copy(big_hbm, cand_v)
        pltpu.sync_copy(dst_hbm.at[wid], dst_v)
        pltpu.sync_copy(alpha_hbm.at[wid], al_v)

        def chunk(j, carry):
            def ek(k, c2):
                sl = pl.ds(k * 16, 16)
                idx = dst_v[j, sl]
                plsc.store_scatter(cand_v, [idx], al_v[j, sl])
                return c2

            lax.fori_loop(0, CH // 16, ek, 0)
            return carry

        lax.fori_loop(0, NCH, chunk, 0)
        pltpu.sync_copy(cand_v, cands_hbm.at[wid])

    RR = 1024
    @functools.partial(
        pl.kernel,
        out_type=jax.ShapeDtypeStruct((NN2,), jnp.float32),
        mesh=mesh,
        compiler_params=pltpu.CompilerParams(needs_layout_passes=False),
        scratch_types=[
            pltpu.VMEM((NW, RR), jnp.float32),
        ],
    )
    def _reduce_rep(cands_hbm, mhat_hbm, r_v):
        wid = lax.axis_index("s") * 2 + lax.axis_index("c")

        @pl.when(wid < NN2 // RR)
        def _():
            cols = pl.ds(wid * RR, RR)
            pltpu.sync_copy(cands_hbm.at[:, cols], r_v)

            def red(w, carry):
                def vm(v, c2):
                    sl = pl.ds(v * 16, 16)
                    r_v[0, sl] = jnp.minimum(r_v[0, sl], r_v[w, sl])
                    return c2

                lax.fori_loop(0, RR // 16, vm, 0)
                return carry

            lax.fori_loop(1, NW, red, 0)
            pltpu.sync_copy(r_v.at[0], mhat_hbm.at[cols])

    @functools.partial(
        pl.kernel,
        out_type=(
            jax.ShapeDtypeStruct((NW, NCH * CH), jnp.float32),
            jax.ShapeDtypeStruct((NW, NN), jnp.float32),
        ),
        mesh=mesh,
        compiler_params=pltpu.CompilerParams(needs_layout_passes=False),
        scratch_types=[
            pltpu.VMEM((NCH, CH), jnp.int32),
            pltpu.VMEM((NCH, CH), jnp.float32),
            pltpu.VMEM((NN2,), jnp.float32),
            pltpu.VMEM((NN,), jnp.float32),
            pltpu.VMEM((NCH * CH,), jnp.float32),
        ],
    )
    def _edge_weights(alpha_hbm, dst_hbm, mhat_hbm, zs_hbm, e_hbm, s_hbm,
                      dst_v, al_v, mh_v, s_v, e_v):
        wid = lax.axis_index("s") * 2 + lax.axis_index("c")
        pltpu.sync_copy(zs_hbm, s_v)
        pltpu.sync_copy(mhat_hbm, mh_v)
        pltpu.sync_copy(dst_hbm.at[wid], dst_v)
        pltpu.sync_copy(alpha_hbm.at[wid], al_v)

        def chunk(j, carry):
            def ek(k, c2):
                sl = pl.ds(k * 16, 16)
                idx = dst_v[j, sl]
                m = plsc.load_gather(mh_v, [idx])
                e = jnp.exp(al_v[j, sl] - m)
                e_v[pl.ds(j * CH + k * 16, 16)] = e
                plsc.addupdate_scatter(s_v, [idx], e)
                return c2

            lax.fori_loop(0, CH // 16, ek, 0, unroll=5)
            return carry

        lax.fori_loop(0, NCH, chunk, 0)
        pltpu.sync_copy(e_v, e_hbm.at[wid])
        pltpu.sync_copy(s_v, s_hbm.at[wid])

    @functools.partial(
        pl.kernel,
        out_type=jax.ShapeDtypeStruct((2, NN2, HH), jnp.float32),
        mesh=mesh,
        compiler_params=pltpu.CompilerParams(needs_layout_passes=False),
        scratch_types=[
            pltpu.VMEM((CPS, CH), jnp.int32),
            pltpu.VMEM((CPS, CH), jnp.int32),
            pltpu.VMEM((NCH * CH,), jnp.float32),
            pltpu.VMEM((2, CH, HH), jnp.float32),
            pltpu.VMEM_SHARED((NN2, HH), jnp.float32),
            pltpu.SemaphoreType.DMA,
            pltpu.SemaphoreType.DMA,
        ],
    )
    def _edge_softmax_scatter(e4_hbm, dst_hbm, src_hbm, feat_hbm,
                              zrows_hbm, outp_hbm,
                              dst_v, src_v, e_v, rows_v,
                              out_sh, gsem, wsem):
        cid = lax.axis_index("c")
        sid = lax.axis_index("s")
        wid = sid * 2 + cid
        myrows = pl.ds(sid * RPT, RPT)

        pltpu.sync_copy(zrows_hbm, rows_v.at[0])

        def zinit(w, carry):
            pltpu.sync_copy(rows_v.at[0],
                            out_sh.at[pl.ds(sid * RPT + w * CH, CH)])
            return carry

        lax.fori_loop(0, RPT // CH, zinit, 0)
        pltpu.sync_copy(e4_hbm.at[wid], e_v)
        plsc.subcore_barrier()

        def seg(g, carry0):
            pltpu.sync_copy(dst_hbm.at[wid, g], dst_v)
            pltpu.sync_copy(src_hbm.at[wid, g], src_v)

            def start(j, p):
                pltpu.async_copy(feat_hbm.at[src_v.at[j]], rows_v.at[p], gsem)

            def wait_g(j, p):
                pltpu.make_async_copy(feat_hbm.at[src_v.at[j]], rows_v.at[p],
                                      gsem).wait()

            def wait_s(j, p):
                pltpu.make_async_copy(rows_v.at[p],
                                      out_sh.at[dst_v.at[j]], wsem).wait()

            start(0, 0)

            def chunk(j, carry):
                p = j % 2
                wait_g(j, p)

                @pl.when(j + 1 < CPS)
                def _():
                    @pl.when(j >= 1)
                    def _():
                        wait_s(j - 1, 1 - p)

                    start(j + 1, 1 - p)

                def row(i, c2):
                    eb = plsc.load_gather(
                        e_v, [jnp.full((16,), (g * CPS + j) * CH + i,
                                       jnp.int32)])
                    for c in range(LB):
                        sl = pl.ds(c * 16, 16)
                        rows_v[p, i, sl] = rows_v[p, i, sl] * eb
                    return c2

                lax.fori_loop(0, CH, row, 0, unroll=4)
                pltpu.async_copy(rows_v.at[p], out_sh.at[dst_v.at[j]],
                                 wsem, add=True)
                return carry

            lax.fori_loop(0, CPS, chunk, 0)
            wait_s(CPS - 2, (CPS - 2) % 2)
            wait_s(CPS - 1, (CPS - 1) % 2)
            return carry0

        lax.fori_loop(0, SEG, seg, 0)
        plsc.subcore_barrier()
        pltpu.sync_copy(out_sh.at[myrows], outp_hbm.at[cid, myrows])

    return (_gather_sum, _scatter_rep, _reduce_rep, _edge_weights,
            _edge_softmax_scatter)


def _upd_proj_body(feat_ref, outp_ref, s_ref, w1_ref, w2_ref,
                   featn_ref, p_ref, q_ref):
    s = jnp.sum(s_ref[...], axis=(0, 1, 2))
    inv = 1.0 / (s + 1e-16)
    fn = feat_ref[...] + (outp_ref[0] + outp_ref[1]) * inv[:, None]
    featn_ref[...] = fn
    p_ref[...] = jnp.dot(fn, w1_ref[...], preferred_element_type=jnp.float32)
    q_ref[...] = jnp.dot(fn, w2_ref[...], preferred_element_type=jnp.float32)


_RB = 1000


def _upd_proj(feat, outp, s_all, w1, w2):
    bs_rows = pl.BlockSpec((_RB, HH), lambda g: (g, 0))
    return pl.pallas_call(
        _upd_proj_body,
        grid=(NN // _RB,),
        in_specs=[
            bs_rows,
            pl.BlockSpec((2, _RB, HH), lambda g: (0, g, 0)),
            pl.BlockSpec((NW, 1, 1, _RB), lambda g: (0, g, 0, 0)),
            pl.BlockSpec((HH, HH), lambda g: (0, 0)),
            pl.BlockSpec((HH, HH), lambda g: (0, 0)),
        ],
        out_specs=[bs_rows, bs_rows, bs_rows],
        out_shape=[
            jax.ShapeDtypeStruct((NN, HH), jnp.float32),
            jax.ShapeDtypeStruct((NN, HH), jnp.float32),
            jax.ShapeDtypeStruct((NN, HH), jnp.float32),
        ],
    )(feat, outp, s_all.reshape(NW, NN // _RB, 1, _RB), w1, w2)


def _leaky(x):
    return jnp.where(x > 0, x, 0.01 * x)


def _mlp_body(x1_ref, geo_ref, w3_ref, bin_ref, wh_ref, bh_ref,
              att_ref, alpha_ref):
    h = (x1_ref[...] + jnp.dot(geo_ref[...], w3_ref[...],
                               preferred_element_type=jnp.float32)
         + bin_ref[...])
    h = jnp.maximum(h, 0.0)
    for l in range(NHID):
        h = jnp.maximum(
            jnp.dot(h, wh_ref[l], preferred_element_type=jnp.float32)
            + bh_ref[l], 0.0)
    h = jnp.maximum(h, 0.01 * h)
    u = h * att_ref[...]
    t = jnp.maximum(u, 0.01 * u)
    ones = jnp.ones((HH, 1), jnp.float32)
    alpha_ref[0, 0, :] = jnp.dot(t, ones,
                                 preferred_element_type=jnp.float32)[:, 0]


def _mlp(x1, geo, w3, b_in, w_hid, b_hid, att):
    bs_edge = pl.BlockSpec((BE, HH), lambda g: (g, 0))
    bs_w = pl.BlockSpec((HH, HH), lambda g: (0, 0))
    alpha3 = pl.pallas_call(
        _mlp_body,
        grid=(GB,),
        in_specs=[
            bs_edge,
            bs_edge,
            bs_w,
            pl.BlockSpec((1, HH), lambda g: (0, 0)),
            pl.BlockSpec((NHID, HH, HH), lambda g: (0, 0, 0)),
            pl.BlockSpec((NHID, HH), lambda g: (0, 0)),
            pl.BlockSpec((1, HH), lambda g: (0, 0)),
        ],
        out_specs=pl.BlockSpec((1, 1, BE), lambda g: (g, 0, 0)),
        out_shape=jax.ShapeDtypeStruct((GB, 1, BE), jnp.float32),
    )(x1, geo, w3, b_in, w_hid, b_hid, att)
    return alpha3.reshape(EE)


def kernel(node_feature, geo_encoding, edge_index, W_in, b_in, W_hid, b_hid, att):
    (gather_sum, scatter_rep, reduce_rep, edge_weights,
     edge_softmax_scatter) = _sc_kernels()
    src2 = edge_index[0].reshape(NW, NCH, CH)
    dst2 = edge_index[1].reshape(NW, NCH, CH)
    zrows = jnp.zeros((CH, HH), jnp.float32)
    zs = jnp.zeros((NN,), jnp.float32)
    big = jnp.full((NN2,), 3e38, jnp.float32)
    outp = jnp.zeros((2, NN2, HH), jnp.float32)
    s_all = jnp.zeros((NW, NN), jnp.float32)

    feat = node_feature
    for blk in range(NBLK):
        feat, p, q = _upd_proj(feat, outp, s_all,
                               W_in[blk, :HH], W_in[blk, HH:2 * HH])
        x1 = gather_sum(p, q, dst2, src2)
        alpha = _mlp(x1, geo_encoding, W_in[blk, 2 * HH:],
                     b_in[blk].reshape(1, HH), W_hid[blk], b_hid[blk],
                     att[blk].reshape(1, HH))
        cands = scatter_rep(alpha.reshape(NW, NCH, CH), dst2, big)
        mhat = reduce_rep(cands)
        e4, s_all = edge_weights(alpha.reshape(NW, NCH, CH), dst2, mhat, zs)
        outp = edge_softmax_scatter(
            e4, dst2.reshape(NW, SEG, CPS, CH),
            src2.reshape(NW, SEG, CPS, CH), feat, zrows)
    feat, _, _ = _upd_proj(feat, outp, s_all, W_in[0, :HH], W_in[0, HH:2 * HH])
    return feat


# speedup vs baseline: 1.1047x; 1.0919x over previous
import functools

import jax
import jax.numpy as jnp
from jax import lax
from jax.experimental import pallas as pl
from jax.experimental.pallas import tpu as pltpu
from jax.experimental.pallas import tpu_sc as plsc

NN = 10000
EE = 320000
HH = 128
NBLK = 3
NHID = 3

NW = 32
EPW = EE // NW
CH = 80
NCH = EPW // CH
NROW = EE // CH
NN2 = 10240
RPT = NN2 // 16
LB = HH // 16

SEG = 5
CPS = NCH // SEG

BE = 4000
GB = EE // BE


@functools.lru_cache(maxsize=None)
def _sc_kernels():
    mesh = plsc.VectorSubcoreMesh(core_axis_name="c", subcore_axis_name="s")

    @functools.partial(
        pl.kernel,
        out_type=jax.ShapeDtypeStruct((EE, HH), jnp.float32),
        mesh=mesh,
        compiler_params=pltpu.CompilerParams(needs_layout_passes=False),
        scratch_types=[
            pltpu.VMEM((NCH, CH), jnp.int32),
            pltpu.VMEM((NCH, CH), jnp.int32),
            pltpu.VMEM((2, CH, HH), jnp.float32),
            pltpu.VMEM((2, CH, HH), jnp.float32),
            pltpu.SemaphoreType.DMA,
            pltpu.SemaphoreType.DMA,
        ],
    )
    def _gather_sum(p_hbm, q_hbm, dst_hbm, src_hbm, x1_hbm,
                    dst_v, src_v, rp_v, rq_v, gsem, wsem):
        wid = lax.axis_index("s") * 2 + lax.axis_index("c")
        ebase = wid * EPW
        pltpu.sync_copy(dst_hbm.at[wid], dst_v)
        pltpu.sync_copy(src_hbm.at[wid], src_v)

        def start(j, p):
            pltpu.async_copy(p_hbm.at[dst_v.at[j]], rp_v.at[p], gsem)
            pltpu.async_copy(q_hbm.at[src_v.at[j]], rq_v.at[p], gsem)

        def wait_g(j, p):
            pltpu.make_async_copy(p_hbm.at[dst_v.at[j]], rp_v.at[p],
                                  gsem).wait()
            pltpu.make_async_copy(q_hbm.at[src_v.at[j]], rq_v.at[p],
                                  gsem).wait()

        def wait_w(j, p):
            pltpu.make_async_copy(
                rp_v.at[p], x1_hbm.at[pl.ds(ebase + j * CH, CH)], wsem).wait()

        start(0, 0)

        def chunk(j, carry):
            p = j % 2
            wait_g(j, p)

            @pl.when(j + 1 < NCH)
            def _():
                @pl.when(j >= 1)
                def _():
                    wait_w(j - 1, 1 - p)

                start(j + 1, 1 - p)

            def row(i, c2):
                for c in range(LB):
                    sl = pl.ds(c * 16, 16)
                    rp_v[p, i, sl] = rp_v[p, i, sl] + rq_v[p, i, sl]
                return c2

            lax.fori_loop(0, CH, row, 0, unroll=4)
            pltpu.async_copy(rp_v.at[p], x1_hbm.at[pl.ds(ebase + j * CH, CH)],
                             wsem)
            return carry

        lax.fori_loop(0, NCH, chunk, 0)
        wait_w(NCH - 2, (NCH - 2) % 2)
        wait_w(NCH - 1, (NCH - 1) % 2)

    @functools.partial(
        pl.kernel,
        out_type=jax.ShapeDtypeStruct((NW, NN2), jnp.float32),
        mesh=mesh,
        compiler_params=pltpu.CompilerParams(needs_layout_passes=False),
        scratch_types=[
            pltpu.VMEM((NCH, CH), jnp.int32),
            pltpu.VMEM((NCH, CH), jnp.float32),
            pltpu.VMEM((NN2,), jnp.float32),
        ],
    )
    def _scatter_rep(alpha_hbm, dst_hbm, big_hbm, cands_hbm, dst_v, al_v,
                     cand_v):
        wid = lax.axis_index("s") * 2 + lax.axis_index("c")
        pltpu.sync_copy(big_hbm, cand_v)
        pltpu.sync_copy(dst_hbm.at[wid], dst_v)
        pltpu.sync_copy(alpha_hbm.at[wid], al_v)

        def chunk(j, carry):
            def ek(k, c2):
                sl = pl.ds(k * 16, 16)
                idx = dst_v[j, sl]
                plsc.store_scatter(cand_v, [idx], al_v[j, sl])
                return c2

            lax.fori_loop(0, CH // 16, ek, 0)
            return carry

        lax.fori_loop(0, NCH, chunk, 0)
        pltpu.sync_copy(cand_v, cands_hbm.at[wid])

    RR = 1024
    @functools.partial(
        pl.kernel,
        out_type=jax.ShapeDtypeStruct((NN2,), jnp.float32),
        mesh=mesh,
        compiler_params=pltpu.CompilerParams(needs_layout_passes=False),
        scratch_types=[
            pltpu.VMEM((NW, RR), jnp.float32),
        ],
    )
    def _reduce_rep(cands_hbm, mhat_hbm, r_v):
        wid = lax.axis_index("s") * 2 + lax.axis_index("c")

        @pl.when(wid < NN2 // RR)
        def _():
            cols = pl.ds(wid * RR, RR)
            pltpu.sync_copy(cands_hbm.at[:, cols], r_v)

            def red(w, carry):
                def vm(v, c2):
                    sl = pl.ds(v * 16, 16)
                    r_v[0, sl] = jnp.minimum(r_v[0, sl], r_v[w, sl])
                    return c2

                lax.fori_loop(0, RR // 16, vm, 0)
                return carry

            lax.fori_loop(1, NW, red, 0)
            pltpu.sync_copy(r_v.at[0], mhat_hbm.at[cols])

    @functools.partial(
        pl.kernel,
        out_type=(
            jax.ShapeDtypeStruct((NW, NCH * CH), jnp.float32),
            jax.ShapeDtypeStruct((NW, NN), jnp.float32),
        ),
        mesh=mesh,
        compiler_params=pltpu.CompilerParams(needs_layout_passes=False),
        scratch_types=[
            pltpu.VMEM((NCH, CH), jnp.int32),
            pltpu.VMEM((NCH, CH), jnp.float32),
            pltpu.VMEM((NN2,), jnp.float32),
            pltpu.VMEM((NN,), jnp.float32),
            pltpu.VMEM((NCH * CH,), jnp.float32),
        ],
    )
    def _edge_weights(alpha_hbm, dst_hbm, mhat_hbm, zs_hbm, e_hbm, s_hbm,
                      dst_v, al_v, mh_v, s_v, e_v):
        wid = lax.axis_index("s") * 2 + lax.axis_index("c")
        pltpu.sync_copy(zs_hbm, s_v)
        pltpu.sync_copy(mhat_hbm, mh_v)
        pltpu.sync_copy(dst_hbm.at[wid], dst_v)
        pltpu.sync_copy(alpha_hbm.at[wid], al_v)

        def chunk(j, carry):
            def ek(k, c2):
                sl = pl.ds(k * 16, 16)
                idx = dst_v[j, sl]
                m = plsc.load_gather(mh_v, [idx])
                e = jnp.exp(al_v[j, sl] - m)
                e_v[pl.ds(j * CH + k * 16, 16)] = e
                plsc.addupdate_scatter(s_v, [idx], e)
                return c2

            lax.fori_loop(0, CH // 16, ek, 0, unroll=5)
            return carry

        lax.fori_loop(0, NCH, chunk, 0)
        pltpu.sync_copy(e_v, e_hbm.at[wid])
        pltpu.sync_copy(s_v, s_hbm.at[wid])

    @functools.partial(
        pl.kernel,
        out_type=jax.ShapeDtypeStruct((2, NN2, HH), jnp.float32),
        mesh=mesh,
        compiler_params=pltpu.CompilerParams(needs_layout_passes=False),
        scratch_types=[
            pltpu.VMEM((CPS, CH), jnp.int32),
            pltpu.VMEM((CPS, CH), jnp.int32),
            pltpu.VMEM((NCH * CH,), jnp.float32),
            pltpu.VMEM((2, CH, HH), jnp.float32),
            pltpu.VMEM_SHARED((NN2, HH), jnp.float32),
            pltpu.SemaphoreType.DMA,
            pltpu.SemaphoreType.DMA,
        ],
    )
    def _edge_softmax_scatter(e4_hbm, dst_hbm, src_hbm, feat_hbm,
                              zrows_hbm, outp_hbm,
                              dst_v, src_v, e_v, rows_v,
                              out_sh, gsem, wsem):
        cid = lax.axis_index("c")
        sid = lax.axis_index("s")
        wid = sid * 2 + cid
        myrows = pl.ds(sid * RPT, RPT)

        pltpu.sync_copy(zrows_hbm, rows_v.at[0])

        def zinit(w, carry):
            pltpu.sync_copy(rows_v.at[0],
                            out_sh.at[pl.ds(sid * RPT + w * CH, CH)])
            return carry

        lax.fori_loop(0, RPT // CH, zinit, 0)
        pltpu.sync_copy(e4_hbm.at[wid], e_v)
        plsc.subcore_barrier()

        def seg(g, carry0):
            pltpu.sync_copy(dst_hbm.at[wid, g], dst_v)
            pltpu.sync_copy(src_hbm.at[wid, g], src_v)

            def start(j, p):
                pltpu.async_copy(feat_hbm.at[src_v.at[j]], rows_v.at[p], gsem)

            def wait_g(j, p):
                pltpu.make_async_copy(feat_hbm.at[src_v.at[j]], rows_v.at[p],
                                      gsem).wait()

            def wait_s(j, p):
                pltpu.make_async_copy(rows_v.at[p],
                                      out_sh.at[dst_v.at[j]], wsem).wait()

            start(0, 0)

            def chunk(j, carry):
                p = j % 2
                wait_g(j, p)

                @pl.when(j + 1 < CPS)
                def _():
                    @pl.when(j >= 1)
                    def _():
                        wait_s(j - 1, 1 - p)

                    start(j + 1, 1 - p)

                def row(i, c2):
                    eb = plsc.load_gather(
                        e_v, [jnp.full((16,), (g * CPS + j) * CH + i,
                                       jnp.int32)])
                    for c in range(LB):
                        sl = pl.ds(c * 16, 16)
                        rows_v[p, i, sl] = rows_v[p, i, sl] * eb
                    return c2

                lax.fori_loop(0, CH, row, 0, unroll=4)
                pltpu.async_copy(rows_v.at[p], out_sh.at[dst_v.at[j]],
                                 wsem, add=True)
                return carry

            lax.fori_loop(0, CPS, chunk, 0)
            wait_s(CPS - 2, (CPS - 2) % 2)
            wait_s(CPS - 1, (CPS - 1) % 2)
            return carry0

        lax.fori_loop(0, SEG, seg, 0)
        plsc.subcore_barrier()
        pltpu.sync_copy(out_sh.at[myrows], outp_hbm.at[cid, myrows])

    return (_gather_sum, _scatter_rep, _reduce_rep, _edge_weights,
            _edge_softmax_scatter)


def _upd_proj_body(feat_ref, outp_ref, s_ref, w1_ref, w2_ref,
                   featn_ref, p_ref, q_ref):
    s = jnp.sum(s_ref[...], axis=(0, 1, 2))
    inv = 1.0 / (s + 1e-16)
    fn = feat_ref[...] + (outp_ref[0] + outp_ref[1]) * inv[:, None]
    featn_ref[...] = fn
    p_ref[...] = jnp.dot(fn, w1_ref[...], preferred_element_type=jnp.float32)
    q_ref[...] = jnp.dot(fn, w2_ref[...], preferred_element_type=jnp.float32)


_RB = 1000


def _upd_proj(feat, outp, s_all, w1, w2):
    bs_rows = pl.BlockSpec((_RB, HH), lambda g: (g, 0))
    return pl.pallas_call(
        _upd_proj_body,
        grid=(NN // _RB,),
        in_specs=[
            bs_rows,
            pl.BlockSpec((2, _RB, HH), lambda g: (0, g, 0)),
            pl.BlockSpec((NW, 1, 1, _RB), lambda g: (0, g, 0, 0)),
            pl.BlockSpec((HH, HH), lambda g: (0, 0)),
            pl.BlockSpec((HH, HH), lambda g: (0, 0)),
        ],
        out_specs=[bs_rows, bs_rows, bs_rows],
        out_shape=[
            jax.ShapeDtypeStruct((NN, HH), jnp.float32),
            jax.ShapeDtypeStruct((NN, HH), jnp.float32),
            jax.ShapeDtypeStruct((NN, HH), jnp.float32),
        ],
    )(feat, outp, s_all.reshape(NW, NN // _RB, 1, _RB), w1, w2)


def _leaky(x):
    return jnp.where(x > 0, x, 0.01 * x)


def _mlp_body(x1_ref, geo_ref, w3_ref, bin_ref, wh_ref, bh_ref,
              att_ref, alpha_ref):
    h = (x1_ref[...] + jnp.dot(geo_ref[...], w3_ref[...],
                               preferred_element_type=jnp.float32)
         + bin_ref[...])
    h = jnp.maximum(h, 0.0)
    for l in range(NHID):
        h = jnp.maximum(
            jnp.dot(h, wh_ref[l], preferred_element_type=jnp.float32)
            + bh_ref[l], 0.0)
    h = jnp.maximum(h, 0.01 * h)
    u = h * att_ref[...]
    t = jnp.maximum(u, 0.01 * u)
    ones = jnp.ones((HH, 1), jnp.float32)
    alpha_ref[0, 0, :] = jnp.dot(t, ones,
                                 preferred_element_type=jnp.float32)[:, 0]


def _mlp(x1, geo, w3, b_in, w_hid, b_hid, att):
    bs_edge = pl.BlockSpec((BE, HH), lambda g: (g, 0))
    bs_w = pl.BlockSpec((HH, HH), lambda g: (0, 0))
    alpha3 = pl.pallas_call(
        _mlp_body,
        grid=(GB,),
        in_specs=[
            bs_edge,
            bs_edge,
            bs_w,
            pl.BlockSpec((1, HH), lambda g: (0, 0)),
            pl.BlockSpec((NHID, HH, HH), lambda g: (0, 0, 0)),
            pl.BlockSpec((NHID, HH), lambda g: (0, 0)),
            pl.BlockSpec((1, HH), lambda g: (0, 0)),
        ],
        out_specs=pl.BlockSpec((1, 1, BE), lambda g: (g, 0, 0)),
        out_shape=jax.ShapeDtypeStruct((GB, 1, BE), jnp.float32),
    )(x1, geo, w3, b_in, w_hid, b_hid, att)
    return alpha3.reshape(EE)


def kernel(node_feature, geo_encoding, edge_index, W_in, b_in, W_hid, b_hid, att):
    (gather_sum, scatter_rep, reduce_rep, edge_weights,
     edge_softmax_scatter) = _sc_kernels()
    src2 = edge_index[0].reshape(NW, NCH, CH)
    dst2 = edge_index[1].reshape(NW, NCH, CH)
    zrows = jnp.zeros((CH, HH), jnp.float32)
    zs = jnp.zeros((NN,), jnp.float32)
    big = jnp.full((NN2,), 3e38, jnp.float32)
    outp = jnp.zeros((2, NN2, HH), jnp.float32)
    s_all = jnp.zeros((NW, NN), jnp.float32)

    feat = node_feature
    for blk in range(NBLK):
        feat, p, q = _upd_proj(feat, outp, s_all,
                               W_in[blk, :HH], W_in[blk, HH:2 * HH])
        x1 = gather_sum(p, q, dst2, src2)
        alpha = _mlp(x1, geo_encoding, W_in[blk, 2 * HH:],
                     b_in[blk].reshape(1, HH), W_hid[blk], b_hid[blk],
                     att[blk].reshape(1, HH))
        cands = scatter_rep(alpha.reshape(NW, NCH, CH), dst2, big)
        mhat = reduce_rep(cands)
        e4, s_all = edge_weights(alpha.reshape(NW, NCH, CH), dst2, mhat, zs)
        outp = edge_softmax_scatter(
            e4, dst2.reshape(NW, SEG, CPS, CH),
            src2.reshape(NW, SEG, CPS, CH), feat, zrows)
    feat, _, _ = _upd_proj(feat, outp, s_all, W_in[0, :HH], W_in[0, HH:2 * HH])
    return feat


# speedup vs baseline: 1.1098x; 1.0047x over previous
import functools

import jax
import jax.numpy as jnp
from jax import lax
from jax.experimental import pallas as pl
from jax.experimental.pallas import tpu as pltpu
from jax.experimental.pallas import tpu_sc as plsc

NN = 10000
EE = 320000
HH = 128
NBLK = 3
NHID = 3

NW = 32
EPW = EE // NW
CH = 80
NCH = EPW // CH
NROW = EE // CH
NN2 = 10240
RPT = NN2 // 16
LB = HH // 16

SEG = 5
CPS = NCH // SEG

BE = 8000
GB = EE // BE


@functools.lru_cache(maxsize=None)
def _sc_kernels():
    mesh = plsc.VectorSubcoreMesh(core_axis_name="c", subcore_axis_name="s")

    @functools.partial(
        pl.kernel,
        out_type=jax.ShapeDtypeStruct((EE, HH), jnp.float32),
        mesh=mesh,
        compiler_params=pltpu.CompilerParams(needs_layout_passes=False),
        scratch_types=[
            pltpu.VMEM((NCH, CH), jnp.int32),
            pltpu.VMEM((NCH, CH), jnp.int32),
            pltpu.VMEM((2, CH, HH), jnp.float32),
            pltpu.VMEM((2, CH, HH), jnp.float32),
            pltpu.SemaphoreType.DMA,
            pltpu.SemaphoreType.DMA,
        ],
    )
    def _gather_sum(p_hbm, q_hbm, dst_hbm, src_hbm, x1_hbm,
                    dst_v, src_v, rp_v, rq_v, gsem, wsem):
        wid = lax.axis_index("s") * 2 + lax.axis_index("c")
        ebase = wid * EPW
        pltpu.sync_copy(dst_hbm.at[wid], dst_v)
        pltpu.sync_copy(src_hbm.at[wid], src_v)

        def start(j, p):
            pltpu.async_copy(p_hbm.at[dst_v.at[j]], rp_v.at[p], gsem)
            pltpu.async_copy(q_hbm.at[src_v.at[j]], rq_v.at[p], gsem)

        def wait_g(j, p):
            pltpu.make_async_copy(p_hbm.at[dst_v.at[j]], rp_v.at[p],
                                  gsem).wait()
            pltpu.make_async_copy(q_hbm.at[src_v.at[j]], rq_v.at[p],
                                  gsem).wait()

        def wait_w(j, p):
            pltpu.make_async_copy(
                rp_v.at[p], x1_hbm.at[pl.ds(ebase + j * CH, CH)], wsem).wait()

        start(0, 0)

        def chunk(j, carry):
            p = j % 2
            wait_g(j, p)

            @pl.when(j + 1 < NCH)
            def _():
                @pl.when(j >= 1)
                def _():
                    wait_w(j - 1, 1 - p)

                start(j + 1, 1 - p)

            def row(i, c2):
                for c in range(LB):
                    sl = pl.ds(c * 16, 16)
                    rp_v[p, i, sl] = rp_v[p, i, sl] + rq_v[p, i, sl]
                return c2

            lax.fori_loop(0, CH, row, 0, unroll=4)
            pltpu.async_copy(rp_v.at[p], x1_hbm.at[pl.ds(ebase + j * CH, CH)],
                             wsem)
            return carry

        lax.fori_loop(0, NCH, chunk, 0)
        wait_w(NCH - 2, (NCH - 2) % 2)
        wait_w(NCH - 1, (NCH - 1) % 2)

    @functools.partial(
        pl.kernel,
        out_type=jax.ShapeDtypeStruct((NW, NN2), jnp.float32),
        mesh=mesh,
        compiler_params=pltpu.CompilerParams(needs_layout_passes=False),
        scratch_types=[
            pltpu.VMEM((NCH, CH), jnp.int32),
            pltpu.VMEM((NCH, CH), jnp.float32),
            pltpu.VMEM((NN2,), jnp.float32),
        ],
    )
    def _scatter_rep(alpha_hbm, dst_hbm, big_hbm, cands_hbm, dst_v, al_v,
                     cand_v):
        wid = lax.axis_index("s") * 2 + lax.axis_index("c")
        pltpu.sync_copy(big_hbm, cand_v)
        pltpu.sync_copy(dst_hbm.at[wid], dst_v)
        pltpu.sync_copy(alpha_hbm.at[wid], al_v)

        def chunk(j, carry):
            def ek(k, c2):
                sl = pl.ds(k * 16, 16)
                idx = dst_v[j, sl]
                plsc.store_scatter(cand_v, [idx], al_v[j, sl])
                return c2

            lax.fori_loop(0, CH // 16, ek, 0)
            return carry

        lax.fori_loop(0, NCH, chunk, 0)
        pltpu.sync_copy(cand_v, cands_hbm.at[wid])

    RR = 1024
    @functools.partial(
        pl.kernel,
        out_type=jax.ShapeDtypeStruct((NN2,), jnp.float32),
        mesh=mesh,
        compiler_params=pltpu.CompilerParams(needs_layout_passes=False),
        scratch_types=[
            pltpu.VMEM((NW, RR), jnp.float32),
        ],
    )
    def _reduce_rep(cands_hbm, mhat_hbm, r_v):
        wid = lax.axis_index("s") * 2 + lax.axis_index("c")

        @pl.when(wid < NN2 // RR)
        def _():
            cols = pl.ds(wid * RR, RR)
            pltpu.sync_copy(cands_hbm.at[:, cols], r_v)

            def red(w, carry):
                def vm(v, c2):
                    sl = pl.ds(v * 16, 16)
                    r_v[0, sl] = jnp.minimum(r_v[0, sl], r_v[w, sl])
                    return c2

                lax.fori_loop(0, RR // 16, vm, 0)
                return carry

            lax.fori_loop(1, NW, red, 0)
            pltpu.sync_copy(r_v.at[0], mhat_hbm.at[cols])

    @functools.partial(
        pl.kernel,
        out_type=(
            jax.ShapeDtypeStruct((NW, NCH * CH), jnp.float32),
            jax.ShapeDtypeStruct((NW, NN), jnp.float32),
        ),
        mesh=mesh,
        compiler_params=pltpu.CompilerParams(needs_layout_passes=False),
        scratch_types=[
            pltpu.VMEM((NCH, CH), jnp.int32),
            pltpu.VMEM((NCH, CH), jnp.float32),
            pltpu.VMEM((NN2,), jnp.float32),
            pltpu.VMEM((NN,), jnp.float32),
            pltpu.VMEM((NCH * CH,), jnp.float32),
        ],
    )
    def _edge_weights(alpha_hbm, dst_hbm, mhat_hbm, zs_hbm, e_hbm, s_hbm,
                      dst_v, al_v, mh_v, s_v, e_v):
        wid = lax.axis_index("s") * 2 + lax.axis_index("c")
        pltpu.sync_copy(zs_hbm, s_v)
        pltpu.sync_copy(mhat_hbm, mh_v)
        pltpu.sync_copy(dst_hbm.at[wid], dst_v)
        pltpu.sync_copy(alpha_hbm.at[wid], al_v)

        def chunk(j, carry):
            def ek(k, c2):
                sl = pl.ds(k * 16, 16)
                idx = dst_v[j, sl]
                m = plsc.load_gather(mh_v, [idx])
                e = jnp.exp(al_v[j, sl] - m)
                e_v[pl.ds(j * CH + k * 16, 16)] = e
                plsc.addupdate_scatter(s_v, [idx], e)
                return c2

            lax.fori_loop(0, CH // 16, ek, 0, unroll=5)
            return carry

        lax.fori_loop(0, NCH, chunk, 0)
        pltpu.sync_copy(e_v, e_hbm.at[wid])
        pltpu.sync_copy(s_v, s_hbm.at[wid])

    @functools.partial(
        pl.kernel,
        out_type=jax.ShapeDtypeStruct((2, NN2, HH), jnp.float32),
        mesh=mesh,
        compiler_params=pltpu.CompilerParams(needs_layout_passes=False),
        scratch_types=[
            pltpu.VMEM((CPS, CH), jnp.int32),
            pltpu.VMEM((CPS, CH), jnp.int32),
            pltpu.VMEM((NCH * CH,), jnp.float32),
            pltpu.VMEM((2, CH, HH), jnp.float32),
            pltpu.VMEM_SHARED((NN2, HH), jnp.float32),
            pltpu.SemaphoreType.DMA,
            pltpu.SemaphoreType.DMA,
        ],
    )
    def _edge_softmax_scatter(e4_hbm, dst_hbm, src_hbm, feat_hbm,
                              zrows_hbm, outp_hbm,
                              dst_v, src_v, e_v, rows_v,
                              out_sh, gsem, wsem):
        cid = lax.axis_index("c")
        sid = lax.axis_index("s")
        wid = sid * 2 + cid
        myrows = pl.ds(sid * RPT, RPT)

        pltpu.sync_copy(zrows_hbm, rows_v.at[0])

        def zinit(w, carry):
            pltpu.sync_copy(rows_v.at[0],
                            out_sh.at[pl.ds(sid * RPT + w * CH, CH)])
            return carry

        lax.fori_loop(0, RPT // CH, zinit, 0)
        pltpu.sync_copy(e4_hbm.at[wid], e_v)
        plsc.subcore_barrier()

        def seg(g, carry0):
            pltpu.sync_copy(dst_hbm.at[wid, g], dst_v)
            pltpu.sync_copy(src_hbm.at[wid, g], src_v)

            def start(j, p):
                pltpu.async_copy(feat_hbm.at[src_v.at[j]], rows_v.at[p], gsem)

            def wait_g(j, p):
                pltpu.make_async_copy(feat_hbm.at[src_v.at[j]], rows_v.at[p],
                                      gsem).wait()

            def wait_s(j, p):
                pltpu.make_async_copy(rows_v.at[p],
                                      out_sh.at[dst_v.at[j]], wsem).wait()

            start(0, 0)

            def chunk(j, carry):
                p = j % 2
                wait_g(j, p)

                @pl.when(j + 1 < CPS)
                def _():
                    @pl.when(j >= 1)
                    def _():
                        wait_s(j - 1, 1 - p)

                    start(j + 1, 1 - p)

                def row(i, c2):
                    eb = plsc.load_gather(
                        e_v, [jnp.full((16,), (g * CPS + j) * CH + i,
                                       jnp.int32)])
                    for c in range(LB):
                        sl = pl.ds(c * 16, 16)
                        rows_v[p, i, sl] = rows_v[p, i, sl] * eb
                    return c2

                lax.fori_loop(0, CH, row, 0, unroll=4)
                pltpu.async_copy(rows_v.at[p], out_sh.at[dst_v.at[j]],
                                 wsem, add=True)
                return carry

            lax.fori_loop(0, CPS, chunk, 0)
            wait_s(CPS - 2, (CPS - 2) % 2)
            wait_s(CPS - 1, (CPS - 1) % 2)
            return carry0

        lax.fori_loop(0, SEG, seg, 0)
        plsc.subcore_barrier()
        pltpu.sync_copy(out_sh.at[myrows], outp_hbm.at[cid, myrows])

    return (_gather_sum, _scatter_rep, _reduce_rep, _edge_weights,
            _edge_softmax_scatter)


def _upd_proj_body(feat_ref, outp_ref, s_ref, w1_ref, w2_ref,
                   featn_ref, p_ref, q_ref):
    s = jnp.sum(s_ref[...], axis=(0, 1, 2))
    inv = 1.0 / (s + 1e-16)
    fn = feat_ref[...] + (outp_ref[0] + outp_ref[1]) * inv[:, None]
    featn_ref[...] = fn
    p_ref[...] = jnp.dot(fn, w1_ref[...], preferred_element_type=jnp.float32)
    q_ref[...] = jnp.dot(fn, w2_ref[...], preferred_element_type=jnp.float32)


_RB = 1000


def _upd_proj(feat, outp, s_all, w1, w2):
    bs_rows = pl.BlockSpec((_RB, HH), lambda g: (g, 0))
    return pl.pallas_call(
        _upd_proj_body,
        grid=(NN // _RB,),
        in_specs=[
            bs_rows,
            pl.BlockSpec((2, _RB, HH), lambda g: (0, g, 0)),
            pl.BlockSpec((NW, 1, 1, _RB), lambda g: (0, g, 0, 0)),
            pl.BlockSpec((HH, HH), lambda g: (0, 0)),
            pl.BlockSpec((HH, HH), lambda g: (0, 0)),
        ],
        out_specs=[bs_rows, bs_rows, bs_rows],
        out_shape=[
            jax.ShapeDtypeStruct((NN, HH), jnp.float32),
            jax.ShapeDtypeStruct((NN, HH), jnp.float32),
            jax.ShapeDtypeStruct((NN, HH), jnp.float32),
        ],
    )(feat, outp, s_all.reshape(NW, NN // _RB, 1, _RB), w1, w2)


def _leaky(x):
    return jnp.where(x > 0, x, 0.01 * x)


def _mlp_body(x1_ref, geo_ref, w3_ref, bin_ref, wh_ref, bh_ref,
              att_ref, alpha_ref):
    h = (x1_ref[...] + jnp.dot(geo_ref[...], w3_ref[...],
                               preferred_element_type=jnp.float32)
         + bin_ref[...])
    h = jnp.maximum(h, 0.0)
    for l in range(NHID):
        h = jnp.maximum(
            jnp.dot(h, wh_ref[l], preferred_element_type=jnp.float32)
            + bh_ref[l], 0.0)
    h = jnp.maximum(h, 0.01 * h)
    u = h * att_ref[...]
    t = jnp.maximum(u, 0.01 * u)
    ones = jnp.ones((HH, 1), jnp.float32)
    alpha_ref[0, 0, :] = jnp.dot(t, ones,
                                 preferred_element_type=jnp.float32)[:, 0]


def _mlp(x1, geo, w3, b_in, w_hid, b_hid, att):
    bs_edge = pl.BlockSpec((BE, HH), lambda g: (g, 0))
    bs_w = pl.BlockSpec((HH, HH), lambda g: (0, 0))
    alpha3 = pl.pallas_call(
        _mlp_body,
        grid=(GB,),
        in_specs=[
            bs_edge,
            bs_edge,
            bs_w,
            pl.BlockSpec((1, HH), lambda g: (0, 0)),
            pl.BlockSpec((NHID, HH, HH), lambda g: (0, 0, 0)),
            pl.BlockSpec((NHID, HH), lambda g: (0, 0)),
            pl.BlockSpec((1, HH), lambda g: (0, 0)),
        ],
        out_specs=pl.BlockSpec((1, 1, BE), lambda g: (g, 0, 0)),
        out_shape=jax.ShapeDtypeStruct((GB, 1, BE), jnp.float32),
    )(x1, geo, w3, b_in, w_hid, b_hid, att)
    return alpha3.reshape(EE)


def kernel(node_feature, geo_encoding, edge_index, W_in, b_in, W_hid, b_hid, att):
    (gather_sum, scatter_rep, reduce_rep, edge_weights,
     edge_softmax_scatter) = _sc_kernels()
    src2 = edge_index[0].reshape(NW, NCH, CH)
    dst2 = edge_index[1].reshape(NW, NCH, CH)
    zrows = jnp.zeros((CH, HH), jnp.float32)
    zs = jnp.zeros((NN,), jnp.float32)
    big = jnp.full((NN2,), 3e38, jnp.float32)
    outp = jnp.zeros((2, NN2, HH), jnp.float32)
    s_all = jnp.zeros((NW, NN), jnp.float32)

    feat = node_feature
    for blk in range(NBLK):
        feat, p, q = _upd_proj(feat, outp, s_all,
                               W_in[blk, :HH], W_in[blk, HH:2 * HH])
        x1 = gather_sum(p, q, dst2, src2)
        alpha = _mlp(x1, geo_encoding, W_in[blk, 2 * HH:],
                     b_in[blk].reshape(1, HH), W_hid[blk], b_hid[blk],
                     att[blk].reshape(1, HH))
        cands = scatter_rep(alpha.reshape(NW, NCH, CH), dst2, big)
        mhat = reduce_rep(cands)
        e4, s_all = edge_weights(alpha.reshape(NW, NCH, CH), dst2, mhat, zs)
        outp = edge_softmax_scatter(
            e4, dst2.reshape(NW, SEG, CPS, CH),
            src2.reshape(NW, SEG, CPS, CH), feat, zrows)
    feat, _, _ = _upd_proj(feat, outp, s_all, W_in[0, :HH], W_in[0, HH:2 * HH])
    return feat
